# baseline (speedup 1.0000x reference)
import functools

import jax
import jax.numpy as jnp
from jax.experimental import pallas as pl
from jax.experimental.pallas import tpu as pltpu


def _gates_body(x4_ref, p5_ref, h_ref, wx_ref, wh_ref, bias_ref, wc2_ref,
                out_ref):
    C = h_ref.shape[-1]
    uh = jnp.concatenate([h_ref[...], p5_ref[4]], axis=1)
    gh = jnp.dot(uh, wh_ref[...], preferred_element_type=jnp.float32)
    gh = gh + bias_ref[...]
    acc = jnp.zeros(out_ref.shape, jnp.float32)
    for p in range(4):
        u = jnp.concatenate([x4_ref[p], p5_ref[p]], axis=1)
        a = jnp.dot(u, wx_ref[...], preferred_element_type=jnp.float32) + gh
        i = jax.nn.sigmoid(a[:, 0:C])
        t = jnp.tanh(a[:, C:2 * C])
        cn = i * t
        o = jax.nn.sigmoid(a[:, 2 * C:3 * C] + wc2_ref[...] * cn)
        acc = acc + o * jnp.tanh(cn)
    out_ref[...] = acc


def _dense_gates(x4, p5, h, wx, wh, bias, wc2):
    n, c = h.shape
    rows = 400
    grid = (n // rows,)
    return pl.pallas_call(
        _gates_body,
        grid=grid,
        in_specs=[
            pl.BlockSpec((4, rows, c), lambda i: (0, i, 0)),
            pl.BlockSpec((5, rows, c), lambda i: (0, i, 0)),
            pl.BlockSpec((rows, c), lambda i: (i, 0)),
            pl.BlockSpec((2 * c, 3 * c), lambda i: (0, 0)),
            pl.BlockSpec((2 * c, 3 * c), lambda i: (0, 0)),
            pl.BlockSpec((1, 3 * c), lambda i: (0, 0)),
            pl.BlockSpec((1, c), lambda i: (0, 0)),
        ],
        out_specs=pl.BlockSpec((rows, c), lambda i: (i, 0)),
        out_shape=jax.ShapeDtypeStruct((n, c), jnp.float32),
    )(x4, p5, h, wx, wh, bias, wc2)


def kernel(X, edge_index, edge_weight, H, Wx0, Wx1, bx, Wh0, Wh1, bh, wc, b):
    n, in_c, periods = X.shape
    out_c = H.shape[1]
    src = edge_index[0]
    dst = edge_index[1]

    deg = jnp.zeros((n,), jnp.float32).at[src].add(edge_weight)
    dinv = jnp.where(deg > 0, jax.lax.rsqrt(jnp.where(deg > 0, deg, 1.0)), 0.0)
    x4 = jnp.transpose(X, (2, 0, 1))
    s = jnp.concatenate([x4, H[None]], axis=0)
    s_scaled = s * dinv[None, :, None]
    p5 = jnp.zeros_like(s).at[:, dst].add(
        -edge_weight[None, :, None] * s_scaled[:, src])
    p5 = p5 * dinv[None, :, None]

    gsel = jnp.array([0, 2, 3])
    wx = jnp.concatenate([Wx0[gsel], Wx1[gsel]], axis=1)
    wx = jnp.concatenate([wx[0], wx[1], wx[2]], axis=1)
    wh = jnp.concatenate([Wh0[gsel], Wh1[gsel]], axis=1)
    wh = jnp.concatenate([wh[0], wh[1], wh[2]], axis=1)
    bias = (bx[gsel] + bh[gsel] + b[gsel]).reshape(1, 3 * out_c)
    wc2 = wc[2].reshape(1, out_c)
    return _dense_gates(x4, p5, H, wx, wh, bias, wc2)


# speedup vs baseline: 11.8548x; 11.8548x over previous
import functools

import jax
import jax.numpy as jnp
from jax import lax
from jax.experimental import pallas as pl
from jax.experimental.pallas import tpu as pltpu
from jax.experimental.pallas import tpu_sc as plsc

NC = 2
NS = 16
NW = NC * NS
B = 128
NB = 80
Q = NB * B
CW = 128


def _deg_body(srcb_hbm, ewb_hbm, out_hbm, srcb_v, ewb_v, z_v, acc, sem):
    cid = lax.axis_index("c")
    sid = lax.axis_index("s")
    wid = cid * NS + sid
    nacc = acc.shape[0]
    rows = nacc // NS
    pltpu.sync_copy(srcb_hbm.at[wid], srcb_v)
    pltpu.sync_copy(ewb_hbm.at[wid], ewb_v)

    @pl.loop(0, z_v.shape[0] // 16)
    def _zero(i):
        z_v[pl.ds(i * 16, 16)] = jnp.zeros((16,), jnp.float32)

    pltpu.sync_copy(z_v, acc.at[pl.ds(sid * rows, rows)])
    plsc.subcore_barrier()

    @pl.loop(0, NB)
    def _scat(j):
        pltpu.sync_copy(ewb_v.at[j], acc.at[srcb_v.at[j]], add=True)

    plsc.subcore_barrier()
    pltpu.sync_copy(acc.at[pl.ds(sid * rows, rows)],
                    out_hbm.at[cid, pl.ds(sid * rows, rows)])


def _deg_call(srcb, ewb, nacc):
    rows = nacc // NS
    mesh = plsc.VectorSubcoreMesh(core_axis_name="c", subcore_axis_name="s")
    return pl.kernel(
        _deg_body,
        out_type=jax.ShapeDtypeStruct((NC, nacc), jnp.float32),
        mesh=mesh,
        scratch_types=[
            pltpu.VMEM((NB, B), jnp.int32),
            pltpu.VMEM((NB, B), jnp.float32),
            pltpu.VMEM((rows,), jnp.float32),
            pltpu.VMEM_SHARED((nacc,), jnp.float32),
            pltpu.SemaphoreType.DMA,
        ],
    )(srcb, ewb)


def _prop_body(n, table_hbm, srcp_hbm, ewp_hbm, dstb_hbm, out_hbm,
               src_v, w_v, dst_v, gidx_v, rows_v, acc, sem):
    cid = lax.axis_index("c")
    sid = lax.axis_index("s")
    wid = cid * NS + sid
    nacc = acc.shape[0]
    rows = nacc // NS

    pltpu.sync_copy(srcp_hbm.at[wid], src_v)
    pltpu.sync_copy(ewp_hbm.at[wid], w_v)
    pltpu.sync_copy(dstb_hbm.at[wid], dst_v)

    for c in range(5):
        @pl.loop(0, B)
        def _zrows(i):
            for f in range(0, CW, 16):
                rows_v[i, pl.ds(f, 16)] = jnp.zeros((16,), jnp.float32)

        for k in range(rows // B):
            pltpu.sync_copy(rows_v,
                            acc.at[pl.ds(sid * rows + k * B, B)])
        plsc.subcore_barrier()

        @pl.loop(0, NB)
        def _batch(j):
            base = j * B
            for k in range(B // 16):
                gsl = pl.ds(base + k * 16, 16)
                gidx_v[pl.ds(k * 16, 16)] = src_v[gsl] + c * n
            pltpu.async_copy(table_hbm.at[gidx_v], rows_v, sem).wait()
            for k in range(B // 16):
                wv16 = w_v[pl.ds(base + k * 16, 16)]
                for l in range(16):
                    e = k * 16 + l
                    ws = jnp.broadcast_to(wv16[l], (16,))
                    for f in range(0, CW, 16):
                        fsl = pl.ds(f, 16)
                        rows_v[e, fsl] = rows_v[e, fsl] * ws
            pltpu.sync_copy(rows_v, acc.at[dst_v.at[j]], add=True)

        plsc.subcore_barrier()
        pltpu.sync_copy(acc.at[pl.ds(sid * rows, rows)],
                        out_hbm.at[cid, c, pl.ds(sid * rows, rows)])
        plsc.subcore_barrier()


def _prop_call(n, table, srcp, ewp, dstb, nacc):
    mesh = plsc.VectorSubcoreMesh(core_axis_name="c", subcore_axis_name="s")
    return pl.kernel(
        functools.partial(_prop_body, n),
        out_type=jax.ShapeDtypeStruct((NC, 5, nacc, CW), jnp.float32),
        mesh=mesh,
        scratch_types=[
            pltpu.VMEM((Q,), jnp.int32),
            pltpu.VMEM((Q,), jnp.float32),
            pltpu.VMEM((NB, B), jnp.int32),
            pltpu.VMEM((B,), jnp.int32),
            pltpu.VMEM((B, CW), jnp.float32),
            pltpu.VMEM_SHARED((nacc, CW), jnp.float32),
            pltpu.SemaphoreType.DMA,
        ],
    )(table, srcp, ewp, dstb)


def _prescale_body(x4_ref, h_ref, dv_ref, out_ref):
    dv = dv_ref[...]
    for c in range(4):
        out_ref[c] = x4_ref[c] * (-dv)
    out_ref[4] = h_ref[...] * (-dv)


def _prescale(x4, h, dv):
    n, c = h.shape
    rows = 400
    return pl.pallas_call(
        _prescale_body,
        grid=(n // rows,),
        in_specs=[
            pl.BlockSpec((4, rows, c), lambda i: (0, i, 0)),
            pl.BlockSpec((rows, c), lambda i: (i, 0)),
            pl.BlockSpec((rows, 1), lambda i: (i, 0)),
        ],
        out_specs=pl.BlockSpec((5, rows, c), lambda i: (0, i, 0)),
        out_shape=jax.ShapeDtypeStruct((5, n, c), jnp.float32),
    )(x4, h, dv)


def _gates_body(x4_ref, pa_ref, pb_ref, h_ref, dv_ref, wx_ref, wh_ref,
                bias_ref, wc2_ref, out_ref):
    C = h_ref.shape[-1]
    dv = dv_ref[...]
    p5 = (pa_ref[0] + pb_ref[0]) * dv[None]
    uh = jnp.concatenate([h_ref[...], p5[4]], axis=1)
    gh = jnp.dot(uh, wh_ref[...], preferred_element_type=jnp.float32)
    gh = gh + bias_ref[...]
    acc = jnp.zeros(out_ref.shape, jnp.float32)
    for p in range(4):
        u = jnp.concatenate([x4_ref[p], p5[p]], axis=1)
        a = jnp.dot(u, wx_ref[...], preferred_element_type=jnp.float32) + gh
        i = jax.nn.sigmoid(a[:, 0:C])
        t = jnp.tanh(a[:, C:2 * C])
        cn = i * t
        o = jax.nn.sigmoid(a[:, 2 * C:3 * C] + wc2_ref[...] * cn)
        acc = acc + o * jnp.tanh(cn)
    out_ref[...] = acc


def _dense_gates(x4, pa, pb, h, dv, wx, wh, bias, wc2):
    n, c = h.shape
    rows = 400
    grid = (n // rows,)
    return pl.pallas_call(
        _gates_body,
        grid=grid,
        in_specs=[
            pl.BlockSpec((4, rows, c), lambda i: (0, i, 0)),
            pl.BlockSpec((1, 5, rows, c), lambda i: (0, 0, i, 0)),
            pl.BlockSpec((1, 5, rows, c), lambda i: (1, 0, i, 0)),
            pl.BlockSpec((rows, c), lambda i: (i, 0)),
            pl.BlockSpec((rows, 1), lambda i: (i, 0)),
            pl.BlockSpec((2 * c, 3 * c), lambda i: (0, 0)),
            pl.BlockSpec((2 * c, 3 * c), lambda i: (0, 0)),
            pl.BlockSpec((1, 3 * c), lambda i: (0, 0)),
            pl.BlockSpec((1, c), lambda i: (0, 0)),
        ],
        out_specs=pl.BlockSpec((rows, c), lambda i: (i, 0)),
        out_shape=jax.ShapeDtypeStruct((n, c), jnp.float32),
    )(x4, pa, pb, h, dv, wx, wh, bias, wc2)


def kernel(X, edge_index, edge_weight, H, Wx0, Wx1, bx, Wh0, Wh1, bh, wc, b):
    n, in_c, periods = X.shape
    out_c = H.shape[1]
    e = edge_index.shape[1]
    src = edge_index[0]
    dst = edge_index[1]

    pad = NW * Q - e
    srcp = jnp.pad(src, (0, pad)).reshape(NW, Q)
    ewp = jnp.pad(edge_weight, (0, pad)).reshape(NW, Q)
    dstb = jnp.pad(dst, (0, pad)).reshape(NW, NB, B)
    srcb = srcp.reshape(NW, NB, B)
    ewb = ewp.reshape(NW, NB, B)

    nacc = ((n + 640 - 1) // 640) * 640

    degp = _deg_call(srcb, ewb, nacc)
    deg = degp[0, :n] + degp[1, :n]
    dinv = jnp.where(deg > 0, lax.rsqrt(jnp.where(deg > 0, deg, 1.0)), 0.0)

    x4 = jnp.transpose(X, (2, 0, 1))
    dv = dinv.reshape(n, 1)
    table = _prescale(x4, H, dv).reshape(5 * n, in_c)

    pp = _prop_call(n, table, srcp, ewp, dstb, nacc)

    gsel = jnp.array([0, 2, 3])
    wx = jnp.concatenate([Wx0[gsel], Wx1[gsel]], axis=1)
    wx = jnp.concatenate([wx[0], wx[1], wx[2]], axis=1)
    wh = jnp.concatenate([Wh0[gsel], Wh1[gsel]], axis=1)
    wh = jnp.concatenate([wh[0], wh[1], wh[2]], axis=1)
    bias = (bx[gsel] + bh[gsel] + b[gsel]).reshape(1, 3 * out_c)
    wc2 = wc[2].reshape(1, out_c)
    return _dense_gates(x4, pp, pp, H, dv, wx, wh, bias, wc2)


# speedup vs baseline: 14.3948x; 1.2143x over previous
import functools

import jax
import jax.numpy as jnp
from jax import lax
from jax.experimental import pallas as pl
from jax.experimental.pallas import tpu as pltpu
from jax.experimental.pallas import tpu_sc as plsc

NC = 2
NS = 16
NW = NC * NS
B = 128
NB = 80
Q = NB * B
CW = 128


def _deg_body(srcb_hbm, ewb_hbm, out_hbm, srcb_v, ewb_v, z_v, acc, sem):
    cid = lax.axis_index("c")
    sid = lax.axis_index("s")
    wid = cid * NS + sid
    nacc = acc.shape[0]
    rows = nacc // NS
    pltpu.sync_copy(srcb_hbm.at[wid], srcb_v)
    pltpu.sync_copy(ewb_hbm.at[wid], ewb_v)

    @pl.loop(0, z_v.shape[0] // 16)
    def _zero(i):
        z_v[pl.ds(i * 16, 16)] = jnp.zeros((16,), jnp.float32)

    pltpu.sync_copy(z_v, acc.at[pl.ds(sid * rows, rows)])
    plsc.subcore_barrier()

    @pl.loop(0, NB)
    def _scat(j):
        pltpu.sync_copy(ewb_v.at[j], acc.at[srcb_v.at[j]], add=True)

    plsc.subcore_barrier()
    pltpu.sync_copy(acc.at[pl.ds(sid * rows, rows)],
                    out_hbm.at[cid, pl.ds(sid * rows, rows)])


def _deg_call(srcb, ewb, nacc):
    rows = nacc // NS
    mesh = plsc.VectorSubcoreMesh(core_axis_name="c", subcore_axis_name="s")
    return pl.kernel(
        _deg_body,
        out_type=jax.ShapeDtypeStruct((NC, nacc), jnp.float32),
        mesh=mesh,
        scratch_types=[
            pltpu.VMEM((NB, B), jnp.int32),
            pltpu.VMEM((NB, B), jnp.float32),
            pltpu.VMEM((rows,), jnp.float32),
            pltpu.VMEM_SHARED((nacc,), jnp.float32),
            pltpu.SemaphoreType.DMA,
        ],
    )(srcb, ewb)


def _prop_body(n, table_hbm, srcp_hbm, ewp_hbm, dstb_hbm, out_hbm,
               src_v, w_v, dst_v, gidx_a, gidx_b, rows_a, rows_b, acc,
               sem_a, sem_b):
    cid = lax.axis_index("c")
    sid = lax.axis_index("s")
    wid = cid * NS + sid
    nacc = acc.shape[0]
    rows = nacc // NS
    HB = NB // 2
    HQ = Q // 2

    def gidx_compute(gv, base, c):
        @pl.loop(0, B // 16)
        def _g(k):
            gv[pl.ds(k * 16, 16)] = src_v[pl.ds(base + k * 16, 16)] + c * n

    def scale(rv, base):
        @pl.loop(0, B // 16)
        def _s(k):
            wv16 = w_v[pl.ds(base + k * 16, 16)]
            for l in range(16):
                e16 = l * 16
                ws = jnp.broadcast_to(wv16[l], (16,))
                row = pl.ds(k * 16 + l, 1)
                for f in range(0, 128, 16):
                    fsl = pl.ds(f, 16)
                    rv[k * 16 + l, fsl] = rv[k * 16 + l, fsl] * ws

    for c in range(5):
        @pl.loop(0, B)
        def _zrows(i):
            for f in range(0, CW, 16):
                rows_a[i, pl.ds(f, 16)] = jnp.zeros((16,), jnp.float32)

        for k in range(rows // B):
            pltpu.sync_copy(rows_a, acc.at[pl.ds(sid * rows + k * B, B)])
        plsc.subcore_barrier()

        for h in range(2):
            pltpu.sync_copy(srcp_hbm.at[wid, pl.ds(h * HQ, HQ)], src_v)
            pltpu.sync_copy(ewp_hbm.at[wid, pl.ds(h * HQ, HQ)], w_v)
            pltpu.sync_copy(dstb_hbm.at[wid, pl.ds(h * HB, HB)], dst_v)
            gidx_compute(gidx_a, 0, c)
            pltpu.async_copy(table_hbm.at[gidx_a], rows_a, sem_a)

            @pl.loop(0, HB // 2)
            def _it(jj):
                base0 = jj * (2 * B)
                b0 = jj * 2
                gidx_compute(gidx_b, base0 + B, c)
                pltpu.make_async_copy(
                    table_hbm.at[gidx_a], rows_a, sem_a).wait()
                pltpu.async_copy(table_hbm.at[gidx_b], rows_b, sem_b)
                scale(rows_a, base0)
                pltpu.sync_copy(rows_a, acc.at[dst_v.at[b0]], add=True)

                @pl.when(jj < HB // 2 - 1)
                def _prefetch():
                    gidx_compute(gidx_a, base0 + 2 * B, c)
                    pltpu.async_copy(table_hbm.at[gidx_a], rows_a, sem_a)

                pltpu.make_async_copy(
                    table_hbm.at[gidx_b], rows_b, sem_b).wait()
                scale(rows_b, base0 + B)
                pltpu.sync_copy(rows_b, acc.at[dst_v.at[b0 + 1]], add=True)

        plsc.subcore_barrier()
        pltpu.sync_copy(acc.at[pl.ds(sid * rows, rows)],
                        out_hbm.at[cid, c, pl.ds(sid * rows, rows)])
        plsc.subcore_barrier()


def _prop_call(n, table, srcp, ewp, dstb, nacc):
    mesh = plsc.VectorSubcoreMesh(core_axis_name="c", subcore_axis_name="s")
    return pl.kernel(
        functools.partial(_prop_body, n),
        out_type=jax.ShapeDtypeStruct((NC, 5, nacc, CW), jnp.float32),
        mesh=mesh,
        scratch_types=[
            pltpu.VMEM((Q // 2,), jnp.int32),
            pltpu.VMEM((Q // 2,), jnp.float32),
            pltpu.VMEM((NB // 2, B), jnp.int32),
            pltpu.VMEM((B,), jnp.int32),
            pltpu.VMEM((B,), jnp.int32),
            pltpu.VMEM((B, CW), jnp.float32),
            pltpu.VMEM((B, CW), jnp.float32),
            pltpu.VMEM_SHARED((nacc, CW), jnp.float32),
            pltpu.SemaphoreType.DMA,
            pltpu.SemaphoreType.DMA,
        ],
    )(table, srcp, ewp, dstb)


def _prescale_body(x4_ref, h_ref, dv_ref, out_ref):
    dv = dv_ref[...]
    for c in range(4):
        out_ref[c] = x4_ref[c] * (-dv)
    out_ref[4] = h_ref[...] * (-dv)


def _prescale(x4, h, dv):
    n, c = h.shape
    rows = 400
    return pl.pallas_call(
        _prescale_body,
        grid=(n // rows,),
        in_specs=[
            pl.BlockSpec((4, rows, c), lambda i: (0, i, 0)),
            pl.BlockSpec((rows, c), lambda i: (i, 0)),
            pl.BlockSpec((rows, 1), lambda i: (i, 0)),
        ],
        out_specs=pl.BlockSpec((5, rows, c), lambda i: (0, i, 0)),
        out_shape=jax.ShapeDtypeStruct((5, n, c), jnp.float32),
    )(x4, h, dv)


def _gates_body(x4_ref, pa_ref, pb_ref, h_ref, dv_ref, wx_ref, wh_ref,
                bias_ref, wc2_ref, out_ref):
    C = h_ref.shape[-1]
    dv = dv_ref[...]
    p5 = (pa_ref[0] + pb_ref[0]) * dv[None]
    uh = jnp.concatenate([h_ref[...], p5[4]], axis=1)
    gh = jnp.dot(uh, wh_ref[...], preferred_element_type=jnp.float32)
    gh = gh + bias_ref[...]
    acc = jnp.zeros(out_ref.shape, jnp.float32)
    for p in range(4):
        u = jnp.concatenate([x4_ref[p], p5[p]], axis=1)
        a = jnp.dot(u, wx_ref[...], preferred_element_type=jnp.float32) + gh
        i = jax.nn.sigmoid(a[:, 0:C])
        t = jnp.tanh(a[:, C:2 * C])
        cn = i * t
        o = jax.nn.sigmoid(a[:, 2 * C:3 * C] + wc2_ref[...] * cn)
        acc = acc + o * jnp.tanh(cn)
    out_ref[...] = acc


def _dense_gates(x4, pa, pb, h, dv, wx, wh, bias, wc2):
    n, c = h.shape
    rows = 400
    grid = (n // rows,)
    return pl.pallas_call(
        _gates_body,
        grid=grid,
        in_specs=[
            pl.BlockSpec((4, rows, c), lambda i: (0, i, 0)),
            pl.BlockSpec((1, 5, rows, c), lambda i: (0, 0, i, 0)),
            pl.BlockSpec((1, 5, rows, c), lambda i: (1, 0, i, 0)),
            pl.BlockSpec((rows, c), lambda i: (i, 0)),
            pl.BlockSpec((rows, 1), lambda i: (i, 0)),
            pl.BlockSpec((2 * c, 3 * c), lambda i: (0, 0)),
            pl.BlockSpec((2 * c, 3 * c), lambda i: (0, 0)),
            pl.BlockSpec((1, 3 * c), lambda i: (0, 0)),
            pl.BlockSpec((1, c), lambda i: (0, 0)),
        ],
        out_specs=pl.BlockSpec((rows, c), lambda i: (i, 0)),
        out_shape=jax.ShapeDtypeStruct((n, c), jnp.float32),
    )(x4, pa, pb, h, dv, wx, wh, bias, wc2)


def kernel(X, edge_index, edge_weight, H, Wx0, Wx1, bx, Wh0, Wh1, bh, wc, b):
    n, in_c, periods = X.shape
    out_c = H.shape[1]
    e = edge_index.shape[1]
    src = edge_index[0]
    dst = edge_index[1]

    pad = NW * Q - e
    srcp = jnp.pad(src, (0, pad)).reshape(NW, Q)
    ewp = jnp.pad(edge_weight, (0, pad)).reshape(NW, Q)
    dstb = jnp.pad(dst, (0, pad)).reshape(NW, NB, B)
    srcb = srcp.reshape(NW, NB, B)
    ewb = ewp.reshape(NW, NB, B)

    nacc = ((n + 640 - 1) // 640) * 640

    degp = _deg_call(srcb, ewb, nacc)
    deg = degp[0, :n] + degp[1, :n]
    dinv = jnp.where(deg > 0, lax.rsqrt(jnp.where(deg > 0, deg, 1.0)), 0.0)

    x4 = jnp.transpose(X, (2, 0, 1))
    dv = dinv.reshape(n, 1)
    table = _prescale(x4, H, dv).reshape(5 * n, in_c)

    pp = _prop_call(n, table, srcp, ewp, dstb, nacc)

    gsel = jnp.array([0, 2, 3])
    wx = jnp.concatenate([Wx0[gsel], Wx1[gsel]], axis=1)
    wx = jnp.concatenate([wx[0], wx[1], wx[2]], axis=1)
    wh = jnp.concatenate([Wh0[gsel], Wh1[gsel]], axis=1)
    wh = jnp.concatenate([wh[0], wh[1], wh[2]], axis=1)
    bias = (bx[gsel] + bh[gsel] + b[gsel]).reshape(1, 3 * out_c)
    wc2 = wc[2].reshape(1, out_c)
    return _dense_gates(x4, pp, pp, H, dv, wx, wh, bias, wc2)


# speedup vs baseline: 15.0039x; 1.0423x over previous
import functools

import jax
import jax.numpy as jnp
from jax import lax
from jax.experimental import pallas as pl
from jax.experimental.pallas import tpu as pltpu
from jax.experimental.pallas import tpu_sc as plsc

NC = 2
NS = 16
NW = NC * NS
B = 64
NB = 160
DEPTH = 4
Q = NB * B
CW = 128


def _deg_body(srcb_hbm, ewb_hbm, out_hbm, srcb_v, ewb_v, z_v, acc, sem):
    cid = lax.axis_index("c")
    sid = lax.axis_index("s")
    wid = cid * NS + sid
    nacc = acc.shape[0]
    rows = nacc // NS
    pltpu.sync_copy(srcb_hbm.at[wid], srcb_v)
    pltpu.sync_copy(ewb_hbm.at[wid], ewb_v)

    @pl.loop(0, z_v.shape[0] // 16)
    def _zero(i):
        z_v[pl.ds(i * 16, 16)] = jnp.zeros((16,), jnp.float32)

    pltpu.sync_copy(z_v, acc.at[pl.ds(sid * rows, rows)])
    plsc.subcore_barrier()

    @pl.loop(0, NB)
    def _scat(j):
        pltpu.sync_copy(ewb_v.at[j], acc.at[srcb_v.at[j]], add=True)

    plsc.subcore_barrier()
    pltpu.sync_copy(acc.at[pl.ds(sid * rows, rows)],
                    out_hbm.at[cid, pl.ds(sid * rows, rows)])


def _deg_call(srcb, ewb, nacc):
    rows = nacc // NS
    mesh = plsc.VectorSubcoreMesh(core_axis_name="c", subcore_axis_name="s")
    return pl.kernel(
        _deg_body,
        out_type=jax.ShapeDtypeStruct((NC, nacc), jnp.float32),
        mesh=mesh,
        scratch_types=[
            pltpu.VMEM((NB, B), jnp.int32),
            pltpu.VMEM((NB, B), jnp.float32),
            pltpu.VMEM((rows,), jnp.float32),
            pltpu.VMEM_SHARED((nacc,), jnp.float32),
            pltpu.SemaphoreType.DMA,
        ],
    )(srcb, ewb)


def _prop_body(n, table_hbm, srcp_hbm, ewp_hbm, dstb_hbm, out_hbm,
               src_v, w_v, dst_v, dstw, gidx, rows, acc, sems):
    cid = lax.axis_index("c")
    sid = lax.axis_index("s")
    wid = cid * NS + sid
    nacc = acc.shape[0]
    arows = nacc // NS
    HB = NB // 2
    HQ = Q // 2

    def gidx_compute(gv, base, c):
        @pl.loop(0, B // 16)
        def _g(k):
            gv[pl.ds(k * 16, 16)] = src_v[pl.ds(base + k * 16, 16)] + c * n

    def scale(rv, base):
        @pl.loop(0, B // 16)
        def _s(k):
            wv16 = w_v[pl.ds(base + k * 16, 16)]
            for l in range(16):
                ws = jnp.broadcast_to(wv16[l], (16,))
                for f in range(0, 128, 16):
                    fsl = pl.ds(f, 16)
                    rv[k * 16 + l, fsl] = rv[k * 16 + l, fsl] * ws

    zb = arows // B

    for c in range(5):
        @pl.loop(0, B)
        def _zrows(i):
            for f in range(0, CW, 16):
                rows[0][i, pl.ds(f, 16)] = jnp.zeros((16,), jnp.float32)

        for k in range(zb):
            pltpu.sync_copy(rows[0], acc.at[pl.ds(sid * arows + k * B, B)])
        plsc.subcore_barrier()

        for h in range(2):
            pltpu.sync_copy(srcp_hbm.at[wid, pl.ds(h * HQ, HQ)], src_v)
            pltpu.sync_copy(ewp_hbm.at[wid, pl.ds(h * HQ, HQ)], w_v)
            pltpu.sync_copy(dstb_hbm.at[wid, pl.ds(h * (HB // 2), HB // 2)],
                            dst_v)
            for b in range(DEPTH):
                gidx_compute(gidx[b], b * B, c)
                pltpu.async_copy(table_hbm.at[gidx[b]], rows[b], sems[b])

            @pl.loop(0, HB // DEPTH)
            def _it(jj):
                for b in range(DEPTH):
                    bi = jj * DEPTH + b
                    base = bi * B
                    pltpu.make_async_copy(
                        table_hbm.at[gidx[b]], rows[b], sems[b]).wait()
                    scale(rows[b], base)
                    j2 = jj * 2 + (b // 2)
                    off = (b % 2) * B
                    for k in range(B // 16):
                        dstw[pl.ds(k * 16, 16)] = (
                            dst_v[j2, pl.ds(off + k * 16, 16)])
                    pltpu.sync_copy(rows[b], acc.at[dstw], add=True)

                    @pl.when(jj < HB // DEPTH - 1)
                    def _prefetch():
                        gidx_compute(gidx[b], base + DEPTH * B, c)
                        pltpu.async_copy(table_hbm.at[gidx[b]], rows[b],
                                         sems[b])

        plsc.subcore_barrier()
        pltpu.sync_copy(acc.at[pl.ds(sid * arows, arows)],
                        out_hbm.at[cid, c, pl.ds(sid * arows, arows)])
        plsc.subcore_barrier()


def _prop_call(n, table, srcp, ewp, dstb, nacc):
    mesh = plsc.VectorSubcoreMesh(core_axis_name="c", subcore_axis_name="s")
    return pl.kernel(
        functools.partial(_prop_body, n),
        out_type=jax.ShapeDtypeStruct((NC, 5, nacc, CW), jnp.float32),
        mesh=mesh,
        scratch_types=[
            pltpu.VMEM((Q // 2,), jnp.int32),
            pltpu.VMEM((Q // 2,), jnp.float32),
            pltpu.VMEM((NB // 4, 2 * B), jnp.int32),
            pltpu.VMEM((B,), jnp.int32),
            [pltpu.VMEM((B,), jnp.int32) for _ in range(DEPTH)],
            [pltpu.VMEM((B, CW), jnp.float32) for _ in range(DEPTH)],
            pltpu.VMEM_SHARED((nacc, CW), jnp.float32),
            [pltpu.SemaphoreType.DMA for _ in range(DEPTH)],
        ],
    )(table, srcp, ewp, dstb)


def _prescale_body(x4_ref, h_ref, dv_ref, out_ref):
    dv = dv_ref[...]
    for c in range(4):
        out_ref[c] = x4_ref[c] * (-dv)
    out_ref[4] = h_ref[...] * (-dv)


def _prescale(x4, h, dv):
    n, c = h.shape
    rows = 400
    return pl.pallas_call(
        _prescale_body,
        grid=(n // rows,),
        in_specs=[
            pl.BlockSpec((4, rows, c), lambda i: (0, i, 0)),
            pl.BlockSpec((rows, c), lambda i: (i, 0)),
            pl.BlockSpec((rows, 1), lambda i: (i, 0)),
        ],
        out_specs=pl.BlockSpec((5, rows, c), lambda i: (0, i, 0)),
        out_shape=jax.ShapeDtypeStruct((5, n, c), jnp.float32),
    )(x4, h, dv)


def _gates_body(x4_ref, pa_ref, pb_ref, h_ref, dv_ref, wx_ref, wh_ref,
                bias_ref, wc2_ref, out_ref):
    C = h_ref.shape[-1]
    dv = dv_ref[...]
    p5 = (pa_ref[0] + pb_ref[0]) * dv[None]
    uh = jnp.concatenate([h_ref[...], p5[4]], axis=1)
    gh = jnp.dot(uh, wh_ref[...], preferred_element_type=jnp.float32)
    gh = gh + bias_ref[...]
    acc = jnp.zeros(out_ref.shape, jnp.float32)
    for p in range(4):
        u = jnp.concatenate([x4_ref[p], p5[p]], axis=1)
        a = jnp.dot(u, wx_ref[...], preferred_element_type=jnp.float32) + gh
        i = jax.nn.sigmoid(a[:, 0:C])
        t = jnp.tanh(a[:, C:2 * C])
        cn = i * t
        o = jax.nn.sigmoid(a[:, 2 * C:3 * C] + wc2_ref[...] * cn)
        acc = acc + o * jnp.tanh(cn)
    out_ref[...] = acc


def _dense_gates(x4, pa, pb, h, dv, wx, wh, bias, wc2):
    n, c = h.shape
    rows = 400
    grid = (n // rows,)
    return pl.pallas_call(
        _gates_body,
        grid=grid,
        in_specs=[
            pl.BlockSpec((4, rows, c), lambda i: (0, i, 0)),
            pl.BlockSpec((1, 5, rows, c), lambda i: (0, 0, i, 0)),
            pl.BlockSpec((1, 5, rows, c), lambda i: (1, 0, i, 0)),
            pl.BlockSpec((rows, c), lambda i: (i, 0)),
            pl.BlockSpec((rows, 1), lambda i: (i, 0)),
            pl.BlockSpec((2 * c, 3 * c), lambda i: (0, 0)),
            pl.BlockSpec((2 * c, 3 * c), lambda i: (0, 0)),
            pl.BlockSpec((1, 3 * c), lambda i: (0, 0)),
            pl.BlockSpec((1, c), lambda i: (0, 0)),
        ],
        out_specs=pl.BlockSpec((rows, c), lambda i: (i, 0)),
        out_shape=jax.ShapeDtypeStruct((n, c), jnp.float32),
    )(x4, pa, pb, h, dv, wx, wh, bias, wc2)


def kernel(X, edge_index, edge_weight, H, Wx0, Wx1, bx, Wh0, Wh1, bh, wc, b):
    n, in_c, periods = X.shape
    out_c = H.shape[1]
    e = edge_index.shape[1]
    src = edge_index[0]
    dst = edge_index[1]

    pad = NW * Q - e
    srcp = jnp.pad(src, (0, pad)).reshape(NW, Q)
    ewp = jnp.pad(edge_weight, (0, pad)).reshape(NW, Q)
    dstb = jnp.pad(dst, (0, pad)).reshape(NW, NB // 2, 2 * B)
    srcb = srcp.reshape(NW, NB, B)
    ewb = ewp.reshape(NW, NB, B)

    nacc = ((n + 640 - 1) // 640) * 640

    degp = _deg_call(srcb, ewb, nacc)
    deg = degp[0, :n] + degp[1, :n]
    dinv = jnp.where(deg > 0, lax.rsqrt(jnp.where(deg > 0, deg, 1.0)), 0.0)

    x4 = jnp.transpose(X, (2, 0, 1))
    dv = dinv.reshape(n, 1)
    table = _prescale(x4, H, dv).reshape(5 * n, in_c)

    pp = _prop_call(n, table, srcp, ewp, dstb, nacc)

    gsel = jnp.array([0, 2, 3])
    wx = jnp.concatenate([Wx0[gsel], Wx1[gsel]], axis=1)
    wx = jnp.concatenate([wx[0], wx[1], wx[2]], axis=1)
    wh = jnp.concatenate([Wh0[gsel], Wh1[gsel]], axis=1)
    wh = jnp.concatenate([wh[0], wh[1], wh[2]], axis=1)
    bias = (bx[gsel] + bh[gsel] + b[gsel]).reshape(1, 3 * out_c)
    wc2 = wc[2].reshape(1, out_c)
    return _dense_gates(x4, pp, pp, H, dv, wx, wh, bias, wc2)
